# baseline (speedup 1.0000x reference)
import functools

import jax
import jax.numpy as jnp
from jax import lax
from jax.experimental import pallas as pl
from jax.experimental.pallas import tpu as pltpu
from jax.experimental.pallas import tpu_sc as plsc

N = 10000
C = 128
H = 128
E = 320000

NC = 2
NS = 16
L = 16
NW = NC * NS

NT = 10240
CW = 64
TOTCH = 5120
EPAD = TOTCH * CW
GRP = 32
NBUF = 4
RPT = NT // NS

EC0 = 320
EC1 = 0
DC0 = 208
DC1 = 112
assert NS * (EC0 + EC1) == TOTCH and NS * (DC0 + DC1) == TOTCH

def _sc_mesh():
    return plsc.VectorSubcoreMesh(
        core_axis_name="c", subcore_axis_name="s",
        num_cores=NC, num_subcores=NS)


def _deg_body(dst_hbm, out_hbm, deg_v, didx_v):
    c = lax.axis_index("c")
    s = lax.axis_index("s")
    base = jnp.where(c == 0, s * DC0, NS * DC0 + s * DC1)
    nch = jnp.where(c == 0, DC0, DC1)
    zeros16 = jnp.zeros((L,), jnp.float32)

    def zb(i, carry):
        deg_v[pl.ds(i * L, L)] = zeros16
        return carry

    lax.fori_loop(0, NT // L, zb, 0)

    pltpu.sync_copy(dst_hbm.at[pl.ds(base, DC1)], didx_v.at[pl.ds(0, DC1)])

    @pl.when(c == 0)
    def _():
        pltpu.sync_copy(dst_hbm.at[pl.ds(base + DC1, DC0 - DC1)],
                        didx_v.at[pl.ds(DC1, DC0 - DC1)])

    ones16 = jnp.ones((L,), jnp.float32)

    def rb(r, carry):
        def kb(k, inner):
            idx = didx_v[r, pl.ds(k * L, L)]
            plsc.addupdate_scatter(deg_v, [idx], ones16)
            return inner

        return lax.fori_loop(0, CW // L, kb, carry)

    lax.fori_loop(0, nch, rb, 0)
    w = s * NC + c
    pltpu.sync_copy(deg_v, out_hbm.at[w])


@functools.cache
def _deg_counts_call():
    return pl.kernel(
        _deg_body,
        out_type=jax.ShapeDtypeStruct((NW, NT), jnp.float32),
        mesh=_sc_mesh(),
        scratch_types=[
            pltpu.VMEM((NT,), jnp.float32),
            pltpu.VMEM((DC0, CW), jnp.int32),
        ],
        compiler_params=pltpu.CompilerParams(needs_layout_passes=False),
    )


def _deg_counts(dst_p):
    return _deg_counts_call()(dst_p)


def _edge_body(src_hbm, dst_hbm, hp_hbm, out_hbm, acc_sh, sidx_v, didx_v,
               rows_v, ssem, isem, *gsems):
    c = lax.axis_index("c")
    s = lax.axis_index("s")
    base = jnp.where(c == 0, s * EC0, NS * EC0 + s * EC1)
    ngrp = jnp.where(c == 0, EC0 // GRP, EC1 // GRP)

    base0 = jnp.minimum(base, TOTCH - GRP)
    di = pltpu.async_copy(src_hbm.at[pl.ds(base0, GRP)], sidx_v.at[0],
                          gsems[0])
    dd = pltpu.async_copy(dst_hbm.at[pl.ds(base0, GRP)], didx_v.at[0], ssem)

    zeros16 = jnp.zeros((L,), jnp.float32)

    def zb(i, carry):
        rows_v[0, i // (H // L), pl.ds((i % (H // L)) * L, L)] = zeros16
        return carry

    lax.fori_loop(0, CW * (H // L), zb, 0)
    for q in range(RPT // CW):
        pltpu.sync_copy(rows_v.at[0], acc_sh.at[pl.ds(s * RPT + q * CW, CW)])
    di.wait()
    dd.wait()
    plsc.subcore_barrier()

    def gb(g, carry):
        bb = lax.rem(g, 2)
        nb = lax.rem(g + 1, 2)
        gg = jnp.minimum(g + 1, ngrp - 1)
        pi = pltpu.async_copy(src_hbm.at[pl.ds(base + gg * GRP, GRP)],
                              sidx_v.at[nb], isem)
        pd = pltpu.async_copy(dst_hbm.at[pl.ds(base + gg * GRP, GRP)],
                              didx_v.at[nb], isem)
        gd = [None] * GRP
        for j in range(NBUF - 1):
            gd[j] = pltpu.async_copy(hp_hbm.at[sidx_v.at[bb, j]],
                                     rows_v.at[j % NBUF], gsems[j % NBUF])
        prev = None
        for j in range(GRP):
            gd[j].wait()
            if prev is not None:
                prev.wait()
            jn = j + NBUF - 1
            if jn < GRP:
                gd[jn] = pltpu.async_copy(hp_hbm.at[sidx_v.at[bb, jn]],
                                          rows_v.at[jn % NBUF],
                                          gsems[jn % NBUF])
            prev = pltpu.async_copy(rows_v.at[j % NBUF],
                                    acc_sh.at[didx_v.at[bb, j]],
                                    ssem, add=True)
        prev.wait()
        pi.wait()
        pd.wait()
        return carry

    lax.fori_loop(0, ngrp, gb, 0)
    plsc.subcore_barrier()
    pltpu.sync_copy(acc_sh.at[pl.ds(s * RPT, RPT)],
                    out_hbm.at[c, pl.ds(s * RPT, RPT)])


@functools.cache
def _edge_scatter_call():
    return pl.kernel(
        _edge_body,
        out_type=jax.ShapeDtypeStruct((NC, NT, H), jnp.float32),
        mesh=_sc_mesh(),
        scratch_types=[
            pltpu.VMEM_SHARED((NT, H), jnp.float32),
            pltpu.VMEM((2, GRP, CW), jnp.int32),
            pltpu.VMEM((2, GRP, CW), jnp.int32),
            pltpu.VMEM((NBUF, CW, H), jnp.float32),
            pltpu.SemaphoreType.DMA,
            pltpu.SemaphoreType.DMA,
        ] + [pltpu.SemaphoreType.DMA] * NBUF,
    )


def _edge_scatter(src_p, dst_p, hp):
    return _edge_scatter_call()(src_p, dst_p, hp)


_F32 = jnp.float32
RB = 512
RD = 1000


def _prep_body(dp_ref, x_ref, w_ref, hp_ref, dinv_ref):
    deg = jnp.sum(dp_ref[...], axis=0) + 1.0
    dinv = lax.rsqrt(deg)
    h = jnp.dot(x_ref[...], w_ref[...], preferred_element_type=_F32)
    hp_ref[...] = h * dinv[:, None]
    dinv_ref[...] = jnp.broadcast_to(dinv[:, None], dinv_ref.shape)


def _prep(deg_parts, x_pad, conv_w):
    grid = NT // RB
    return pl.pallas_call(
        _prep_body,
        grid=(grid,),
        in_specs=[
            pl.BlockSpec((NW, RB), lambda i: (0, i)),
            pl.BlockSpec((RB, C), lambda i: (i, 0)),
            pl.BlockSpec((C, H), lambda i: (0, 0)),
        ],
        out_specs=[
            pl.BlockSpec((RB, H), lambda i: (i, 0)),
            pl.BlockSpec((RB, H), lambda i: (i, 0)),
        ],
        out_shape=[
            jax.ShapeDtypeStruct((NT, H), _F32),
            jax.ShapeDtypeStruct((NT, H), _F32),
        ],
    )(deg_parts, x_pad, conv_w)


def _accum_stats(st_ref, vals, i):
    su = jnp.sum(vals, axis=0)
    ss = jnp.sum(vals * vals, axis=0)
    st = jnp.concatenate([su[None], ss[None], jnp.zeros((6, H), _F32)], axis=0)

    @pl.when(i == 0)
    def _():
        st_ref[...] = st

    @pl.when(i > 0)
    def _():
        st_ref[...] = st_ref[...] + st


def _bn_affine(st_ref, g_ref, b_ref):
    mean = st_ref[0] / N
    var = st_ref[1] / N - mean * mean
    a = g_ref[0] * lax.rsqrt(var + 1e-5)
    c = b_ref[0] - mean * a
    return a, c


def _stage1_body(acc_ref, hp_ref, dinv_ref, x_ref, cb_ref, gw_ref, gb_ref,
                 lw_ref, lb_ref, l2w_ref, l2b_ref, p2_ref, st_ref):
    i = pl.program_id(0)
    agg = acc_ref[0] + acc_ref[1] + hp_ref[...]
    z = jnp.tanh(dinv_ref[...] * agg + cb_ref[...])
    g = jax.nn.sigmoid(
        jnp.dot(z, gw_ref[...], preferred_element_type=_F32) + gb_ref[...])
    xl = jnp.dot(x_ref[...], lw_ref[...], preferred_element_type=_F32) \
        + lb_ref[...]
    mix = (1.0 - g) * xl + g * z
    p2 = jnp.maximum(
        jnp.dot(mix, l2w_ref[...], preferred_element_type=_F32)
        + l2b_ref[...], 0.0)
    p2_ref[...] = p2
    _accum_stats(st_ref, p2, i)


def _stage2_body(p2_ref, st2_ref, g2_ref, b2_ref, l3w_ref, l3b_ref,
                 t1_ref, p3_ref, st3_ref):
    i = pl.program_id(0)
    a, c = _bn_affine(st2_ref, g2_ref, b2_ref)
    t1 = p2_ref[...] * a + c
    t1_ref[...] = t1
    p3 = jnp.maximum(
        jnp.dot(t1, l3w_ref[...], preferred_element_type=_F32)
        + l3b_ref[...], 0.0)
    p3_ref[...] = p3
    _accum_stats(st3_ref, p3, i)


def _stage3_body(p3_ref, st3_ref, t1_ref, g3_ref, b3_ref, l4w_ref, l4b_ref,
                 s_ref, p4_ref, st4_ref):
    i = pl.program_id(0)
    a, c = _bn_affine(st3_ref, g3_ref, b3_ref)
    s = p3_ref[...] * a + c + t1_ref[...]
    s_ref[...] = s
    p4 = jnp.maximum(
        jnp.dot(s, l4w_ref[...], preferred_element_type=_F32)
        + l4b_ref[...], 0.0)
    p4_ref[...] = p4
    _accum_stats(st4_ref, p4, i)


def _stage4_body(p4_ref, st4_ref, s_ref, g4_ref, b4_ref, out_ref):
    a, c = _bn_affine(st4_ref, g4_ref, b4_ref)
    out_ref[...] = p4_ref[...] * a + c + s_ref[...]


def _row_spec(i_map=lambda i: (i, 0)):
    return pl.BlockSpec((RD, H), i_map)


def _full_spec(shape):
    return pl.BlockSpec(shape, lambda i: tuple(0 for _ in shape))


_ST_SHAPE = (8, H)


def _stage1(accs, hp, dinv_b, x, cb, gw, gb, lw, lb, l2w, l2b):
    grid = N // RD
    return pl.pallas_call(
        _stage1_body,
        grid=(grid,),
        in_specs=[
            pl.BlockSpec((NC, RD, H), lambda i: (0, i, 0)),
            _row_spec(), _row_spec(), _row_spec(),
            _full_spec((1, H)), _full_spec((H, H)), _full_spec((1, H)),
            _full_spec((C, H)), _full_spec((1, H)),
            _full_spec((H, H)), _full_spec((1, H)),
        ],
        out_specs=[_row_spec(), _full_spec(_ST_SHAPE)],
        out_shape=[
            jax.ShapeDtypeStruct((N, H), _F32),
            jax.ShapeDtypeStruct(_ST_SHAPE, _F32),
        ],
    )(accs, hp, dinv_b, x, cb, gw, gb, lw, lb, l2w, l2b)


def _stage2(p2, st2, g2, b2, l3w, l3b):
    grid = N // RD
    return pl.pallas_call(
        _stage2_body,
        grid=(grid,),
        in_specs=[
            _row_spec(), _full_spec(_ST_SHAPE),
            _full_spec((1, H)), _full_spec((1, H)),
            _full_spec((H, H)), _full_spec((1, H)),
        ],
        out_specs=[_row_spec(), _row_spec(), _full_spec(_ST_SHAPE)],
        out_shape=[
            jax.ShapeDtypeStruct((N, H), _F32),
            jax.ShapeDtypeStruct((N, H), _F32),
            jax.ShapeDtypeStruct(_ST_SHAPE, _F32),
        ],
    )(p2, st2, g2, b2, l3w, l3b)


def _stage3(p3, st3, t1, g3, b3, l4w, l4b):
    grid = N // RD
    return pl.pallas_call(
        _stage3_body,
        grid=(grid,),
        in_specs=[
            _row_spec(), _full_spec(_ST_SHAPE), _row_spec(),
            _full_spec((1, H)), _full_spec((1, H)),
            _full_spec((H, H)), _full_spec((1, H)),
        ],
        out_specs=[_row_spec(), _row_spec(), _full_spec(_ST_SHAPE)],
        out_shape=[
            jax.ShapeDtypeStruct((N, H), _F32),
            jax.ShapeDtypeStruct((N, H), _F32),
            jax.ShapeDtypeStruct(_ST_SHAPE, _F32),
        ],
    )(p3, st3, t1, g3, b3, l4w, l4b)


def _stage4(p4, st4, s, g4, b4):
    grid = N // RD
    return pl.pallas_call(
        _stage4_body,
        grid=(grid,),
        in_specs=[
            _row_spec(), _full_spec(_ST_SHAPE), _row_spec(),
            _full_spec((1, H)), _full_spec((1, H)),
        ],
        out_specs=_row_spec(),
        out_shape=jax.ShapeDtypeStruct((N, H), _F32),
    )(p4, st4, s, g4, b4)


def kernel(x, edge_index, conv_w, conv_b, lin_w, lin_b, gate_w, gate_b,
           lin2_w, lin2_b, lin3_w, lin3_b, lin4_w, lin4_b,
           bn2_g, bn2_b, bn3_g, bn3_b, bn4_g, bn4_b):
    src = edge_index[0]
    dst = edge_index[1]
    fill = jnp.full((EPAD - E,), N, jnp.int32)
    src_p = jnp.concatenate([src, fill]).reshape(TOTCH, CW)
    dst_p = jnp.concatenate([dst, fill]).reshape(TOTCH, CW)
    x_pad = jnp.concatenate([x, jnp.zeros((NT - N, C), _F32)], axis=0)

    deg_parts = _deg_counts(dst_p)
    hp, dinv_b = _prep(deg_parts, x_pad, conv_w)
    accs = _edge_scatter(src_p, dst_p, hp)

    cb = conv_b.reshape(1, H)
    gb = gate_b.reshape(1, H)
    lb = lin_b.reshape(1, H)
    l2b = lin2_b.reshape(1, H)
    l3b = lin3_b.reshape(1, H)
    l4b = lin4_b.reshape(1, H)

    p2, st2 = _stage1(accs, hp, dinv_b, x, cb, gate_w, gb, lin_w, lb,
                      lin2_w, l2b)
    t1, p3, st3 = _stage2(p2, st2, bn2_g.reshape(1, H), bn2_b.reshape(1, H),
                          lin3_w, l3b)
    s, p4, st4 = _stage3(p3, st3, t1, bn3_g.reshape(1, H), bn3_b.reshape(1, H),
                         lin4_w, l4b)
    return _stage4(p4, st4, s, bn4_g.reshape(1, H), bn4_b.reshape(1, H))


# speedup vs baseline: 3.1109x; 3.1109x over previous
import functools

import jax
import jax.numpy as jnp
from jax import lax
from jax.experimental import pallas as pl
from jax.experimental.pallas import tpu as pltpu
from jax.experimental.pallas import tpu_sc as plsc

N = 10000
C = 128
H = 128
E = 320000

NC = 2
NS = 16
L = 16
NW = NC * NS

NT = 10240
CW = 64
TOTCH = 5120
EPAD = TOTCH * CW
GRP = 32
NBUF = 4
RPT = NT // NS

EC0 = 160
EC1 = 160
DC0 = 160
DC1 = 160
assert NS * (EC0 + EC1) == TOTCH and NS * (DC0 + DC1) == TOTCH

def _sc_mesh():
    return plsc.VectorSubcoreMesh(
        core_axis_name="c", subcore_axis_name="s",
        num_cores=NC, num_subcores=NS)


def _deg_body(dst_hbm, out_hbm, deg_v, didx_v):
    c = lax.axis_index("c")
    s = lax.axis_index("s")
    base = jnp.where(c == 0, s * DC0, NS * DC0 + s * DC1)
    nch = jnp.where(c == 0, DC0, DC1)
    zeros16 = jnp.zeros((L,), jnp.float32)

    def zb(i, carry):
        deg_v[pl.ds(i * L, L)] = zeros16
        return carry

    lax.fori_loop(0, NT // L, zb, 0)

    pltpu.sync_copy(dst_hbm.at[pl.ds(base, DC1)], didx_v.at[pl.ds(0, DC1)])

    if DC0 > DC1:
        @pl.when(c == 0)
        def _():
            pltpu.sync_copy(dst_hbm.at[pl.ds(base + DC1, DC0 - DC1)],
                            didx_v.at[pl.ds(DC1, DC0 - DC1)])

    ones16 = jnp.ones((L,), jnp.float32)

    def rb(r, carry):
        def kb(k, inner):
            idx = didx_v[r, pl.ds(k * L, L)]
            plsc.addupdate_scatter(deg_v, [idx], ones16)
            return inner

        return lax.fori_loop(0, CW // L, kb, carry)

    lax.fori_loop(0, nch, rb, 0)
    w = s * NC + c
    pltpu.sync_copy(deg_v, out_hbm.at[w])


@functools.cache
def _deg_counts_call():
    return pl.kernel(
        _deg_body,
        out_type=jax.ShapeDtypeStruct((NW, NT), jnp.float32),
        mesh=_sc_mesh(),
        scratch_types=[
            pltpu.VMEM((NT,), jnp.float32),
            pltpu.VMEM((DC0, CW), jnp.int32),
        ],
        compiler_params=pltpu.CompilerParams(needs_layout_passes=False),
    )


def _deg_counts(dst_p):
    return _deg_counts_call()(dst_p)


def _edge_body(src_hbm, dst_hbm, hp_hbm, out_hbm, acc_sh, sidx_v, didx_v,
               rows_v, ssem, isem, *gsems):
    c = lax.axis_index("c")
    s = lax.axis_index("s")
    base = jnp.where(c == 0, s * EC0, NS * EC0 + s * EC1)
    ngrp = jnp.where(c == 0, EC0 // GRP, EC1 // GRP)

    base0 = jnp.minimum(base, TOTCH - GRP)
    di = pltpu.async_copy(src_hbm.at[pl.ds(base0, GRP)], sidx_v.at[0],
                          gsems[0])
    dd = pltpu.async_copy(dst_hbm.at[pl.ds(base0, GRP)], didx_v.at[0], ssem)

    zeros16 = jnp.zeros((L,), jnp.float32)

    def zb(i, carry):
        rows_v[0, i // (H // L), pl.ds((i % (H // L)) * L, L)] = zeros16
        return carry

    lax.fori_loop(0, CW * (H // L), zb, 0)
    for q in range(RPT // CW):
        pltpu.sync_copy(rows_v.at[0], acc_sh.at[pl.ds(s * RPT + q * CW, CW)])
    di.wait()
    dd.wait()
    plsc.subcore_barrier()

    def gb(g, carry):
        bb = lax.rem(g, 2)
        nb = lax.rem(g + 1, 2)
        gg = jnp.minimum(g + 1, ngrp - 1)
        pi = pltpu.async_copy(src_hbm.at[pl.ds(base + gg * GRP, GRP)],
                              sidx_v.at[nb], isem)
        pd = pltpu.async_copy(dst_hbm.at[pl.ds(base + gg * GRP, GRP)],
                              didx_v.at[nb], isem)
        gd = [None] * GRP
        for j in range(NBUF - 1):
            gd[j] = pltpu.async_copy(hp_hbm.at[sidx_v.at[bb, j]],
                                     rows_v.at[j % NBUF], gsems[j % NBUF])
        prev = None
        for j in range(GRP):
            gd[j].wait()
            if prev is not None:
                prev.wait()
            jn = j + NBUF - 1
            if jn < GRP:
                gd[jn] = pltpu.async_copy(hp_hbm.at[sidx_v.at[bb, jn]],
                                          rows_v.at[jn % NBUF],
                                          gsems[jn % NBUF])
            prev = pltpu.async_copy(rows_v.at[j % NBUF],
                                    acc_sh.at[didx_v.at[bb, j]],
                                    ssem, add=True)
        prev.wait()
        pi.wait()
        pd.wait()
        return carry

    lax.fori_loop(0, ngrp, gb, 0)
    plsc.subcore_barrier()
    pltpu.sync_copy(acc_sh.at[pl.ds(s * RPT, RPT)],
                    out_hbm.at[c, pl.ds(s * RPT, RPT)])


@functools.cache
def _edge_scatter_call():
    return pl.kernel(
        _edge_body,
        out_type=jax.ShapeDtypeStruct((NC, NT, H), jnp.float32),
        mesh=_sc_mesh(),
        scratch_types=[
            pltpu.VMEM_SHARED((NT, H), jnp.float32),
            pltpu.VMEM((2, GRP, CW), jnp.int32),
            pltpu.VMEM((2, GRP, CW), jnp.int32),
            pltpu.VMEM((NBUF, CW, H), jnp.float32),
            pltpu.SemaphoreType.DMA,
            pltpu.SemaphoreType.DMA,
        ] + [pltpu.SemaphoreType.DMA] * NBUF,
    )


def _edge_scatter(src_p, dst_p, hp):
    return _edge_scatter_call()(src_p, dst_p, hp)


_F32 = jnp.float32
RB = 512
RD = 1000


def _prep_body(dp_ref, x_ref, w_ref, hp_ref, dinv_ref):
    deg = jnp.sum(dp_ref[...], axis=0) + 1.0
    dinv = lax.rsqrt(deg)
    h = jnp.dot(x_ref[...], w_ref[...], preferred_element_type=_F32)
    hp_ref[...] = h * dinv[:, None]
    dinv_ref[...] = jnp.broadcast_to(dinv[:, None], dinv_ref.shape)


def _prep(deg_parts, x_pad, conv_w):
    grid = NT // RB
    return pl.pallas_call(
        _prep_body,
        grid=(grid,),
        in_specs=[
            pl.BlockSpec((NW, RB), lambda i: (0, i)),
            pl.BlockSpec((RB, C), lambda i: (i, 0)),
            pl.BlockSpec((C, H), lambda i: (0, 0)),
        ],
        out_specs=[
            pl.BlockSpec((RB, H), lambda i: (i, 0)),
            pl.BlockSpec((RB, H), lambda i: (i, 0)),
        ],
        out_shape=[
            jax.ShapeDtypeStruct((NT, H), _F32),
            jax.ShapeDtypeStruct((NT, H), _F32),
        ],
    )(deg_parts, x_pad, conv_w)


def _accum_stats(st_ref, vals, i):
    su = jnp.sum(vals, axis=0)
    ss = jnp.sum(vals * vals, axis=0)
    st = jnp.concatenate([su[None], ss[None], jnp.zeros((6, H), _F32)], axis=0)

    @pl.when(i == 0)
    def _():
        st_ref[...] = st

    @pl.when(i > 0)
    def _():
        st_ref[...] = st_ref[...] + st


def _bn_affine(st_ref, g_ref, b_ref):
    mean = st_ref[0] / N
    var = st_ref[1] / N - mean * mean
    a = g_ref[0] * lax.rsqrt(var + 1e-5)
    c = b_ref[0] - mean * a
    return a, c


def _stage1_body(acc_ref, hp_ref, dinv_ref, x_ref, cb_ref, gw_ref, gb_ref,
                 lw_ref, lb_ref, l2w_ref, l2b_ref, p2_ref, st_ref):
    i = pl.program_id(0)
    agg = acc_ref[0] + acc_ref[1] + hp_ref[...]
    z = jnp.tanh(dinv_ref[...] * agg + cb_ref[...])
    g = jax.nn.sigmoid(
        jnp.dot(z, gw_ref[...], preferred_element_type=_F32) + gb_ref[...])
    xl = jnp.dot(x_ref[...], lw_ref[...], preferred_element_type=_F32) \
        + lb_ref[...]
    mix = (1.0 - g) * xl + g * z
    p2 = jnp.maximum(
        jnp.dot(mix, l2w_ref[...], preferred_element_type=_F32)
        + l2b_ref[...], 0.0)
    p2_ref[...] = p2
    _accum_stats(st_ref, p2, i)


def _stage2_body(p2_ref, st2_ref, g2_ref, b2_ref, l3w_ref, l3b_ref,
                 t1_ref, p3_ref, st3_ref):
    i = pl.program_id(0)
    a, c = _bn_affine(st2_ref, g2_ref, b2_ref)
    t1 = p2_ref[...] * a + c
    t1_ref[...] = t1
    p3 = jnp.maximum(
        jnp.dot(t1, l3w_ref[...], preferred_element_type=_F32)
        + l3b_ref[...], 0.0)
    p3_ref[...] = p3
    _accum_stats(st3_ref, p3, i)


def _stage3_body(p3_ref, st3_ref, t1_ref, g3_ref, b3_ref, l4w_ref, l4b_ref,
                 s_ref, p4_ref, st4_ref):
    i = pl.program_id(0)
    a, c = _bn_affine(st3_ref, g3_ref, b3_ref)
    s = p3_ref[...] * a + c + t1_ref[...]
    s_ref[...] = s
    p4 = jnp.maximum(
        jnp.dot(s, l4w_ref[...], preferred_element_type=_F32)
        + l4b_ref[...], 0.0)
    p4_ref[...] = p4
    _accum_stats(st4_ref, p4, i)


def _stage4_body(p4_ref, st4_ref, s_ref, g4_ref, b4_ref, out_ref):
    a, c = _bn_affine(st4_ref, g4_ref, b4_ref)
    out_ref[...] = p4_ref[...] * a + c + s_ref[...]


def _row_spec(i_map=lambda i: (i, 0)):
    return pl.BlockSpec((RD, H), i_map)


def _full_spec(shape):
    return pl.BlockSpec(shape, lambda i: tuple(0 for _ in shape))


_ST_SHAPE = (8, H)


def _stage1(accs, hp, dinv_b, x, cb, gw, gb, lw, lb, l2w, l2b):
    grid = N // RD
    return pl.pallas_call(
        _stage1_body,
        grid=(grid,),
        in_specs=[
            pl.BlockSpec((NC, RD, H), lambda i: (0, i, 0)),
            _row_spec(), _row_spec(), _row_spec(),
            _full_spec((1, H)), _full_spec((H, H)), _full_spec((1, H)),
            _full_spec((C, H)), _full_spec((1, H)),
            _full_spec((H, H)), _full_spec((1, H)),
        ],
        out_specs=[_row_spec(), _full_spec(_ST_SHAPE)],
        out_shape=[
            jax.ShapeDtypeStruct((N, H), _F32),
            jax.ShapeDtypeStruct(_ST_SHAPE, _F32),
        ],
    )(accs, hp, dinv_b, x, cb, gw, gb, lw, lb, l2w, l2b)


def _stage2(p2, st2, g2, b2, l3w, l3b):
    grid = N // RD
    return pl.pallas_call(
        _stage2_body,
        grid=(grid,),
        in_specs=[
            _row_spec(), _full_spec(_ST_SHAPE),
            _full_spec((1, H)), _full_spec((1, H)),
            _full_spec((H, H)), _full_spec((1, H)),
        ],
        out_specs=[_row_spec(), _row_spec(), _full_spec(_ST_SHAPE)],
        out_shape=[
            jax.ShapeDtypeStruct((N, H), _F32),
            jax.ShapeDtypeStruct((N, H), _F32),
            jax.ShapeDtypeStruct(_ST_SHAPE, _F32),
        ],
    )(p2, st2, g2, b2, l3w, l3b)


def _stage3(p3, st3, t1, g3, b3, l4w, l4b):
    grid = N // RD
    return pl.pallas_call(
        _stage3_body,
        grid=(grid,),
        in_specs=[
            _row_spec(), _full_spec(_ST_SHAPE), _row_spec(),
            _full_spec((1, H)), _full_spec((1, H)),
            _full_spec((H, H)), _full_spec((1, H)),
        ],
        out_specs=[_row_spec(), _row_spec(), _full_spec(_ST_SHAPE)],
        out_shape=[
            jax.ShapeDtypeStruct((N, H), _F32),
            jax.ShapeDtypeStruct((N, H), _F32),
            jax.ShapeDtypeStruct(_ST_SHAPE, _F32),
        ],
    )(p3, st3, t1, g3, b3, l4w, l4b)


def _stage4(p4, st4, s, g4, b4):
    grid = N // RD
    return pl.pallas_call(
        _stage4_body,
        grid=(grid,),
        in_specs=[
            _row_spec(), _full_spec(_ST_SHAPE), _row_spec(),
            _full_spec((1, H)), _full_spec((1, H)),
        ],
        out_specs=_row_spec(),
        out_shape=jax.ShapeDtypeStruct((N, H), _F32),
    )(p4, st4, s, g4, b4)


def kernel(x, edge_index, conv_w, conv_b, lin_w, lin_b, gate_w, gate_b,
           lin2_w, lin2_b, lin3_w, lin3_b, lin4_w, lin4_b,
           bn2_g, bn2_b, bn3_g, bn3_b, bn4_g, bn4_b):
    src = edge_index[0]
    dst = edge_index[1]
    fill = N + jnp.arange(EPAD - E, dtype=jnp.int32) % (NT - N)
    src_p = jnp.concatenate([src, fill]).reshape(TOTCH, CW)
    dst_p = jnp.concatenate([dst, fill]).reshape(TOTCH, CW)
    x_pad = jnp.concatenate([x, jnp.zeros((NT - N, C), _F32)], axis=0)

    deg_parts = _deg_counts(dst_p)
    hp, dinv_b = _prep(deg_parts, x_pad, conv_w)
    accs = _edge_scatter(src_p, dst_p, hp)

    cb = conv_b.reshape(1, H)
    gb = gate_b.reshape(1, H)
    lb = lin_b.reshape(1, H)
    l2b = lin2_b.reshape(1, H)
    l3b = lin3_b.reshape(1, H)
    l4b = lin4_b.reshape(1, H)

    p2, st2 = _stage1(accs, hp, dinv_b, x, cb, gate_w, gb, lin_w, lb,
                      lin2_w, l2b)
    t1, p3, st3 = _stage2(p2, st2, bn2_g.reshape(1, H), bn2_b.reshape(1, H),
                          lin3_w, l3b)
    s, p4, st4 = _stage3(p3, st3, t1, bn3_g.reshape(1, H), bn3_b.reshape(1, H),
                         lin4_w, l4b)
    return _stage4(p4, st4, s, bn4_g.reshape(1, H), bn4_b.reshape(1, H))
